# baseline (speedup 1.0000x reference)
import functools

import jax
import jax.numpy as jnp
from jax import lax
from jax.experimental import pallas as pl
from jax.experimental.pallas import tpu as pltpu
from jax.experimental.pallas import tpu_sc as plsc

_NC, _NS = 2, 16
_NW = _NC * _NS
_CHUNK = 1024


def kernel(memory, input_logits):
    size, dim = memory.shape
    num = input_logits.shape[0]
    bulk = size - num
    nfull = bulk // _CHUNK
    rem = bulk - nfull * _CHUNK
    niter = nfull // _NW
    assert nfull % _NW == 0 and num % _NW == 0
    lchunk = num // _NW

    mesh = plsc.VectorSubcoreMesh(core_axis_name="c", subcore_axis_name="s")

    @functools.partial(
        pl.kernel,
        out_type=jax.ShapeDtypeStruct((size, dim), memory.dtype),
        mesh=mesh,
        compiler_params=pltpu.CompilerParams(use_tc_tiling_on_sc=False),
        scratch_types=[
            pltpu.VMEM((_CHUNK, dim), jnp.float32),
            pltpu.VMEM((lchunk, dim), jnp.float32),
        ],
    )
    def run(mem_hbm, log_hbm, out_hbm, buf, lbuf):
        w = lax.axis_index("s") * _NC + lax.axis_index("c")

        lbase = w * lchunk
        pltpu.sync_copy(log_hbm.at[pl.ds(lbase, lchunk)], lbuf)
        pltpu.sync_copy(lbuf, out_hbm.at[pl.ds(lbase, lchunk)])

        def body(j, carry):
            start = num + (w + _NW * j) * _CHUNK
            pltpu.sync_copy(mem_hbm.at[pl.ds(start, _CHUNK)], buf)
            pltpu.sync_copy(buf, out_hbm.at[pl.ds(start, _CHUNK)])
            return carry

        lax.fori_loop(0, niter, body, 0)

        if rem:
            @pl.when(w == 0)
            def _():
                start = num + nfull * _CHUNK
                pltpu.sync_copy(mem_hbm.at[pl.ds(start, rem)],
                                buf.at[pl.ds(0, rem)])
                pltpu.sync_copy(buf.at[pl.ds(0, rem)],
                                out_hbm.at[pl.ds(start, rem)])

    memory_new = run(memory, input_logits)
    new_index = jnp.array(num % size, dtype=jnp.int32)
    return (memory_new, new_index)


# speedup vs baseline: 1.1734x; 1.1734x over previous
import functools

import jax
import jax.numpy as jnp
from jax import lax
from jax.experimental import pallas as pl
from jax.experimental.pallas import tpu as pltpu
from jax.experimental.pallas import tpu_sc as plsc

_NC, _NS = 2, 16
_NW = _NC * _NS
_CHUNK = 768


def kernel(memory, input_logits):
    size, dim = memory.shape
    num = input_logits.shape[0]
    bulk = size - num
    nfull = bulk // _CHUNK
    rem = bulk - nfull * _CHUNK
    niter = nfull // _NW
    assert nfull % _NW == 0 and num % _NW == 0
    lchunk = num // _NW
    assert lchunk <= _CHUNK and rem <= _CHUNK

    mesh = plsc.VectorSubcoreMesh(core_axis_name="c", subcore_axis_name="s")

    @functools.partial(
        pl.kernel,
        out_type=jax.ShapeDtypeStruct((size, dim), memory.dtype),
        mesh=mesh,
        compiler_params=pltpu.CompilerParams(use_tc_tiling_on_sc=True),
        scratch_types=[
            pltpu.VMEM((_CHUNK, dim), jnp.float32),
        ],
    )
    def run(mem_hbm, log_hbm, out_hbm, buf):
        w = lax.axis_index("s") * _NC + lax.axis_index("c")

        lbase = w * lchunk
        pltpu.sync_copy(log_hbm.at[pl.ds(lbase, lchunk)],
                        buf.at[pl.ds(0, lchunk)])
        pltpu.sync_copy(buf.at[pl.ds(0, lchunk)],
                        out_hbm.at[pl.ds(lbase, lchunk)])

        def body(j, carry):
            start = num + (w + _NW * j) * _CHUNK
            pltpu.sync_copy(mem_hbm.at[pl.ds(start, _CHUNK)], buf)
            pltpu.sync_copy(buf, out_hbm.at[pl.ds(start, _CHUNK)])
            return carry

        lax.fori_loop(0, niter, body, 0)

        if rem:
            @pl.when(w == 0)
            def _():
                start = num + nfull * _CHUNK
                pltpu.sync_copy(mem_hbm.at[pl.ds(start, rem)],
                                buf.at[pl.ds(0, rem)])
                pltpu.sync_copy(buf.at[pl.ds(0, rem)],
                                out_hbm.at[pl.ds(start, rem)])

    memory_new = run(memory, input_logits)
    new_index = jnp.array(num % size, dtype=jnp.int32)
    return (memory_new, new_index)


# speedup vs baseline: 1.9118x; 1.6293x over previous
import jax
import jax.numpy as jnp
from jax.experimental import pallas as pl
from jax.experimental.pallas import tpu as pltpu


def kernel(memory, input_logits):
    size, dim = memory.shape
    num = input_logits.shape[0]

    def body(mem_ref, logits_ref, out_ref, idx_ref):
        out_ref[...] = logits_ref[...]
        idx_ref[0] = jnp.int32(num % size)

    memory_new, new_index = pl.pallas_call(
        body,
        grid=(1,),
        in_specs=[
            pl.BlockSpec(memory_space=pl.ANY),
            pl.BlockSpec((num, dim), lambda i: (0, 0)),
        ],
        out_specs=[
            pl.BlockSpec((num, dim), lambda i: (0, 0)),
            pl.BlockSpec(memory_space=pltpu.SMEM),
        ],
        out_shape=[
            jax.ShapeDtypeStruct((size, dim), memory.dtype),
            jax.ShapeDtypeStruct((1,), jnp.int32),
        ],
        input_output_aliases={0: 0},
    )(memory, input_logits)
    return (memory_new, new_index[0])


# speedup vs baseline: 14.0774x; 7.3635x over previous
import jax
import jax.numpy as jnp
from jax.experimental import pallas as pl
from jax.experimental.pallas import tpu as pltpu

_BLOCK = 65536


def kernel(memory, input_logits):
    size, dim = memory.shape
    num = input_logits.shape[0]
    assert num <= _BLOCK
    memt = memory.T
    logt = input_logits.T
    grid = (pl.cdiv(size, _BLOCK),)

    def body(mem_ref, logits_ref, out_ref, idx_ref):
        i = pl.program_id(0)

        @pl.when(i == 0)
        def _():
            out_ref[:, 0:num] = logits_ref[...]
            out_ref[:, num:_BLOCK] = mem_ref[:, num:_BLOCK]
            idx_ref[0] = jnp.int32(num % size)

        @pl.when(i > 0)
        def _():
            out_ref[...] = mem_ref[...]

    outt, new_index = pl.pallas_call(
        body,
        grid=grid,
        in_specs=[
            pl.BlockSpec((dim, _BLOCK), lambda i: (0, i)),
            pl.BlockSpec((dim, num), lambda i: (0, 0)),
        ],
        out_specs=[
            pl.BlockSpec((dim, _BLOCK), lambda i: (0, i)),
            pl.BlockSpec(memory_space=pltpu.SMEM),
        ],
        out_shape=[
            jax.ShapeDtypeStruct((dim, size), memory.dtype),
            jax.ShapeDtypeStruct((1,), jnp.int32),
        ],
    )(memt, logt)
    return (outt.T, new_index[0])


# speedup vs baseline: 14.1461x; 1.0049x over previous
import jax
import jax.numpy as jnp
from jax.experimental import pallas as pl
from jax.experimental.pallas import tpu as pltpu

_BLOCK = 98304


def kernel(memory, input_logits):
    size, dim = memory.shape
    num = input_logits.shape[0]
    assert num <= _BLOCK
    memt = memory.T
    logt = input_logits.T
    grid = (pl.cdiv(size, _BLOCK),)

    def body(mem_ref, logits_ref, out_ref, idx_ref):
        i = pl.program_id(0)

        @pl.when(i == 0)
        def _():
            out_ref[:, 0:num] = logits_ref[...]
            out_ref[:, num:_BLOCK] = mem_ref[:, num:_BLOCK]
            idx_ref[0] = jnp.int32(num % size)

        @pl.when(i > 0)
        def _():
            out_ref[...] = mem_ref[...]

    outt, new_index = pl.pallas_call(
        body,
        grid=grid,
        in_specs=[
            pl.BlockSpec((dim, _BLOCK), lambda i: (0, i)),
            pl.BlockSpec((dim, num), lambda i: (0, 0)),
        ],
        out_specs=[
            pl.BlockSpec((dim, _BLOCK), lambda i: (0, i)),
            pl.BlockSpec(memory_space=pltpu.SMEM),
        ],
        out_shape=[
            jax.ShapeDtypeStruct((dim, size), memory.dtype),
            jax.ShapeDtypeStruct((1,), jnp.int32),
        ],
    )(memt, logt)
    return (outt.T, new_index[0])
